# Initial kernel scaffold; baseline (speedup 1.0000x reference)
import jax
import jax.numpy as jnp
from jax.experimental import pallas as pl


def kernel(emb_user, emb_item, W_user, b_user, W_item, b_item, user, item_p, item_n, edge_index):
    raise NotImplementedError("write your pallas kernel here")


# speedup vs baseline: 3.2047x; 3.2047x over previous
import functools

import jax
import jax.numpy as jnp
from jax import lax
from jax.experimental import pallas as pl
from jax.experimental.pallas import tpu as pltpu
from jax.experimental.pallas import tpu_sc as plsc

NU = 50000
D = 64
HD = 32
E = 800000
B = 4096
NC = 2
NS = 16
CH = 128
NCHUNK = E // CH
QCONV = (NCHUNK + NS - 1) // NS
HCHUNK = NCHUNK // NC
QDEG = (HCHUNK + NS - 1) // NS
PDEG = 51200
DSLAB = PDEG // NS
NUP = 50176
RSLAB = NUP // NS
FCH = 392
NF = RSLAB // FCH
RB2 = 2000
NB2 = NU // RB2
RB7 = 256
NB7 = B // RB7
ALPHA = 0.001
BETA = 0.001

_f32 = jnp.float32
_i32 = jnp.int32


def _sc_mesh():
    return plsc.VectorSubcoreMesh(
        core_axis_name="c", subcore_axis_name="s", num_cores=NC, num_subcores=NS
    )


def _sc_params():
    return pltpu.CompilerParams(use_tc_tiling_on_sc=False)


def _deg_body(edge, zeros_d, ones_c, out, acc, zbuf, obuf, idx, fbuf):
    c = lax.axis_index("c")
    s = lax.axis_index("s")
    pltpu.sync_copy(zeros_d, zbuf)
    pltpu.sync_copy(ones_c, obuf)
    pltpu.sync_copy(zbuf, acc.at[pl.ds(s * DSLAB, DSLAB)])
    plsc.subcore_barrier()
    start = s * QDEG
    n = jnp.clip(HCHUNK - start, 0, QDEG)

    def body(k, carry):
        e0 = (c * HCHUNK + start + k) * CH
        pltpu.sync_copy(edge.at[1, pl.ds(e0, CH)], idx)
        pltpu.sync_copy(obuf, acc.at[idx], add=True)
        return carry

    lax.fori_loop(0, n, body, 0)
    plsc.subcore_barrier()
    pltpu.sync_copy(acc.at[pl.ds(s * DSLAB, DSLAB)], fbuf)
    pltpu.sync_copy(fbuf, out.at[c, pl.ds(s * DSLAB, DSLAB)])


def _make_deg():
    return pl.kernel(
        _deg_body,
        compiler_params=_sc_params(),
        out_type=[jax.ShapeDtypeStruct((NC, PDEG), _f32)],
        mesh=_sc_mesh(),
        scratch_types=[
            pltpu.VMEM_SHARED((PDEG,), _f32),
            pltpu.VMEM((DSLAB,), _f32),
            pltpu.VMEM((CH,), _f32),
            pltpu.VMEM((CH,), _i32),
            pltpu.VMEM((DSLAB,), _f32),
        ],
    )


def _prep_body(deg_ref, eu_ref, ei_ref,
               norm_ref, xul_ref, xuh_ref, xil_ref, xih_ref):
    d = deg_ref[:, 0] + deg_ref[:, 1]
    nrm = lax.rsqrt(jnp.maximum(d, 1.0))
    norm_ref[...] = nrm[:, None]
    xu = eu_ref[...] * nrm[:, None]
    xul_ref[...] = xu[:, :HD]
    xuh_ref[...] = xu[:, HD:]
    xi = ei_ref[...] * nrm[:, None]
    xil_ref[...] = xi[:, :HD]
    xih_ref[...] = xi[:, HD:]


def _prep(deg_t, emb_user, emb_item):
    return pl.pallas_call(
        _prep_body,
        grid=(NB2,),
        in_specs=[
            pl.BlockSpec((RB2, NC), lambda k: (k, 0)),
            pl.BlockSpec((RB2, D), lambda k: (k, 0)),
            pl.BlockSpec((RB2, D), lambda k: (k, 0)),
        ],
        out_specs=[
            pl.BlockSpec((RB2, 1), lambda k: (k, 0)),
            pl.BlockSpec((RB2, HD), lambda k: (k, 0)),
            pl.BlockSpec((RB2, HD), lambda k: (k, 0)),
            pl.BlockSpec((RB2, HD), lambda k: (k, 0)),
            pl.BlockSpec((RB2, HD), lambda k: (k, 0)),
        ],
        out_shape=[
            jax.ShapeDtypeStruct((NUP, 1), _f32),
            jax.ShapeDtypeStruct((NUP, HD), _f32),
            jax.ShapeDtypeStruct((NUP, HD), _f32),
            jax.ShapeDtypeStruct((NUP, HD), _f32),
            jax.ShapeDtypeStruct((NUP, HD), _f32),
        ],
    )(deg_t, emb_user, emb_item)


def _conv_body(edge, a_lo, a_hi, b_lo, b_hi, zeros_f,
               oal, oah, obl, obh,
               acc, sidx, didx, rbuf, fbuf):
    c = lax.axis_index("c")
    s = lax.axis_index("s")
    start = s * QCONV
    n = jnp.clip(NCHUNK - start, 0, QCONV)

    def run_pass(in_lo, in_hi, out_lo, out_hi):
        pltpu.sync_copy(zeros_f, fbuf)
        for f in range(NF):
            pltpu.sync_copy(fbuf, acc.at[pl.ds(s * RSLAB + f * FCH, FCH), :])
        plsc.subcore_barrier()

        def body(k, carry):
            e0 = (start + k) * CH
            pltpu.sync_copy(edge.at[0, pl.ds(e0, CH)], sidx)
            pltpu.sync_copy(edge.at[1, pl.ds(e0, CH)], didx)

            @pl.when(c == 0)
            def _():
                pltpu.sync_copy(in_lo.at[sidx], rbuf)

            @pl.when(c == 1)
            def _():
                pltpu.sync_copy(in_hi.at[sidx], rbuf)

            pltpu.sync_copy(rbuf, acc.at[didx], add=True)
            return carry

        lax.fori_loop(0, n, body, 0)
        plsc.subcore_barrier()
        for f in range(NF):
            r0 = s * RSLAB + f * FCH
            pltpu.sync_copy(acc.at[pl.ds(r0, FCH), :], fbuf)

            @pl.when(c == 0)
            def _():
                pltpu.sync_copy(fbuf, out_lo.at[pl.ds(r0, FCH), :])

            @pl.when(c == 1)
            def _():
                pltpu.sync_copy(fbuf, out_hi.at[pl.ds(r0, FCH), :])

    run_pass(a_lo, a_hi, oal, oah)
    run_pass(b_lo, b_hi, obl, obh)


def _make_conv():
    return pl.kernel(
        _conv_body,
        compiler_params=_sc_params(),
        out_type=[jax.ShapeDtypeStruct((NUP, HD), _f32)] * 4,
        mesh=_sc_mesh(),
        scratch_types=[
            pltpu.VMEM_SHARED((NUP, HD), _f32),
            pltpu.VMEM((CH,), _i32),
            pltpu.VMEM((CH,), _i32),
            pltpu.VMEM((CH, HD), _f32),
            pltpu.VMEM((FCH, HD), _f32),
        ],
    )


def _scale_body(norm_ref, aul_ref, auh_ref, ail_ref, aih_ref,
                yul_ref, yuh_ref, yil_ref, yih_ref):
    n2 = norm_ref[...] * norm_ref[...]
    yul_ref[...] = aul_ref[...] * n2
    yuh_ref[...] = auh_ref[...] * n2
    yil_ref[...] = ail_ref[...] * n2
    yih_ref[...] = aih_ref[...] * n2


def _scale(norm, aul, auh, ail, aih):
    bs = pl.BlockSpec((RB2, HD), lambda k: (k, 0))
    return pl.pallas_call(
        _scale_body,
        grid=(NB2,),
        in_specs=[pl.BlockSpec((RB2, 1), lambda k: (k, 0)), bs, bs, bs, bs],
        out_specs=[bs, bs, bs, bs],
        out_shape=[jax.ShapeDtypeStruct((NUP, HD), _f32)] * 4,
    )(norm, aul, auh, ail, aih)


def _gather_body(user, item_p, item_n, emb_user, emb_item,
                 aul, auh, ail, aih, bul, buh, bil, bih, norm,
                 ue, ual, uah, ubl, ubh, un,
                 pe, pal, pah, pbl, pbh, pn,
                 ne, nal, nah, nbl, nbh, nn,
                 idx, rbuf, hbuf, nbuf):
    c = lax.axis_index("c")
    s = lax.axis_index("s")
    w = s * NC + c
    base = w * CH

    def do_set(iv, embt, tal, tah, tbl, tbh,
               oe, oal_, oah_, obl_, obh_, on_):
        pltpu.sync_copy(iv.at[pl.ds(base, CH)], idx)
        pltpu.sync_copy(embt.at[idx], rbuf)
        pltpu.sync_copy(rbuf, oe.at[pl.ds(base, CH), :])
        pltpu.sync_copy(tal.at[idx], hbuf)
        pltpu.sync_copy(hbuf, oal_.at[pl.ds(base, CH), :])
        pltpu.sync_copy(tah.at[idx], hbuf)
        pltpu.sync_copy(hbuf, oah_.at[pl.ds(base, CH), :])
        pltpu.sync_copy(tbl.at[idx], hbuf)
        pltpu.sync_copy(hbuf, obl_.at[pl.ds(base, CH), :])
        pltpu.sync_copy(tbh.at[idx], hbuf)
        pltpu.sync_copy(hbuf, obh_.at[pl.ds(base, CH), :])
        pltpu.sync_copy(norm.at[idx], nbuf)
        pltpu.sync_copy(nbuf, on_.at[pl.ds(base, CH), :])

    do_set(user, emb_user, ail, aih, bul, buh, ue, ual, uah, ubl, ubh, un)
    do_set(item_p, emb_item, aul, auh, bil, bih, pe, pal, pah, pbl, pbh, pn)
    do_set(item_n, emb_item, aul, auh, bil, bih, ne, nal, nah, nbl, nbh, nn)


def _make_gather():
    row = jax.ShapeDtypeStruct((B, D), _f32)
    half = jax.ShapeDtypeStruct((B, HD), _f32)
    vec = jax.ShapeDtypeStruct((B, 1), _f32)
    return pl.kernel(
        _gather_body,
        compiler_params=_sc_params(),
        out_type=[row, half, half, half, half, vec] * 3,
        mesh=_sc_mesh(),
        scratch_types=[
            pltpu.VMEM((CH,), _i32),
            pltpu.VMEM((CH, D), _f32),
            pltpu.VMEM((CH, HD), _f32),
            pltpu.VMEM((CH, 1), _f32),
        ],
    )


def _sig(x):
    return 1.0 / (1.0 + jnp.exp(-x))


def _combine(e_ref, al_ref, ah_ref, bl_ref, bh_ref, n_ref):
    nrm = n_ref[...]
    lo = (e_ref[...][:, :HD] + nrm * (al_ref[...] + bl_ref[...])) / 3.0
    hi = (e_ref[...][:, HD:] + nrm * (ah_ref[...] + bh_ref[...])) / 3.0
    return lo, hi


def _scores_body(ue, ual, uah, ubl, ubh, un,
                 pe, pal, pah, pbl, pbh, pn,
                 ne, nal, nah, nbl, nbh, nn,
                 wu_ref, bu_ref, wi_ref, bi_ref,
                 s_ref, t_ref, a_ref, c_ref, lsm_ref):
    uf_lo, uf_hi = _combine(ue, ual, uah, ubl, ubh, un)
    ip_lo, ip_hi = _combine(pe, pal, pah, pbl, pbh, pn)
    in_lo, in_hi = _combine(ne, nal, nah, nbl, nbh, nn)
    s_ref[...] = (jnp.sum(uf_lo * ip_lo, axis=1)
                  + jnp.sum(uf_hi * ip_hi, axis=1)) / float(D)
    t_ref[...] = (jnp.sum(uf_lo * in_lo, axis=1)
                  + jnp.sum(uf_hi * in_hi, axis=1)) / float(D)
    wu = wu_ref[0, :]
    wi = wi_ref[0, :]
    us = (jnp.sum(uf_lo * wu[None, :HD], axis=1)
          + jnp.sum(uf_hi * wu[None, HD:], axis=1) + bu_ref[0])
    pi = (jnp.sum(ip_lo * wi[None, :HD], axis=1)
          + jnp.sum(ip_hi * wi[None, HD:], axis=1) + bi_ref[0])
    ni = (jnp.sum(in_lo * wi[None, :HD], axis=1)
          + jnp.sum(in_hi * wi[None, HD:], axis=1) + bi_ref[0])
    sig_u = _sig(us)
    a_ref[...] = _sig(pi) * sig_u
    c_ref[...] = _sig(ni) * sig_u
    l_item = -jnp.mean(jnp.log(_sig(pi) + 1e-10) + jnp.log(1.0 - _sig(ni) + 1e-10))
    l_user = -jnp.mean(jnp.log(sig_u + 1e-10) + jnp.log(1.0 - sig_u + 1e-10))
    lsm_ref[...] = jnp.full((1, 1), ALPHA * l_item + BETA * l_user, _f32)


def _scores(gathered, W_user, b_user, W_item, b_item):
    full_row = pl.BlockSpec((B, D), lambda: (0, 0))
    full_half = pl.BlockSpec((B, HD), lambda: (0, 0))
    full_vec = pl.BlockSpec((B, 1), lambda: (0, 0))
    set_specs = [full_row, full_half, full_half, full_half, full_half, full_vec]
    return pl.pallas_call(
        _scores_body,
        in_specs=set_specs * 3 + [
            pl.BlockSpec((1, D), lambda: (0, 0)),
            pl.BlockSpec(memory_space=pltpu.SMEM),
            pl.BlockSpec((1, D), lambda: (0, 0)),
            pl.BlockSpec(memory_space=pltpu.SMEM),
        ],
        out_specs=[pl.BlockSpec((B,), lambda: (0,))] * 4
        + [pl.BlockSpec((1, 1), lambda: (0, 0))],
        out_shape=[jax.ShapeDtypeStruct((B,), _f32)] * 4
        + [jax.ShapeDtypeStruct((1, 1), _f32)],
    )(*gathered, W_user.reshape(1, D), b_user, W_item.reshape(1, D), b_item)


def _loss_body(s_ref, t_ref, a_ref, c_ref, lsm_ref, out_ref, acc_ref):
    i = pl.program_id(0)

    @pl.when(i == 0)
    def _():
        acc_ref[0] = 0.0

    x = s_ref[...][None, :] * a_ref[...][:, None]
    y = t_ref[...][None, :] * c_ref[...][:, None]
    part = jnp.sum(jnp.log(_sig(x) + 1e-10) + jnp.log(1.0 - _sig(y) + 1e-10))
    acc_ref[0] = acc_ref[0] + part

    @pl.when(i == NB7 - 1)
    def _():
        out_ref[...] = jnp.full(
            (1, 1), -(acc_ref[0] / float(B * B)) + lsm_ref[0, 0], _f32)


def _loss(s, t, a, c, lsm):
    return pl.pallas_call(
        _loss_body,
        grid=(NB7,),
        in_specs=[
            pl.BlockSpec((B,), lambda i: (0,)),
            pl.BlockSpec((B,), lambda i: (0,)),
            pl.BlockSpec((RB7,), lambda i: (i,)),
            pl.BlockSpec((RB7,), lambda i: (i,)),
            pl.BlockSpec((1, 1), lambda i: (0, 0)),
        ],
        out_specs=pl.BlockSpec((1, 1), lambda i: (0, 0)),
        out_shape=jax.ShapeDtypeStruct((1, 1), _f32),
        scratch_shapes=[pltpu.SMEM((1,), _f32)],
    )(s, t, a, c, lsm)


def kernel(emb_user, emb_item, W_user, b_user, W_item, b_item,
           user, item_p, item_n, edge_index):
    edge = edge_index.astype(_i32)
    user = user.astype(_i32)
    item_p = item_p.astype(_i32)
    item_n = item_n.astype(_i32)

    zeros_d = jnp.zeros((DSLAB,), _f32)
    ones_c = jnp.ones((CH,), _f32)
    zeros_f = jnp.zeros((FCH, HD), _f32)

    (deg2,) = _make_deg()(edge, zeros_d, ones_c)
    deg_t = jnp.swapaxes(deg2, 0, 1)
    norm, xul, xuh, xil, xih = _prep(deg_t, emb_user, emb_item)
    conv = _make_conv()
    aul, auh, ail, aih = conv(edge, xul, xuh, xil, xih, zeros_f)
    yul, yuh, yil, yih = _scale(norm, aul, auh, ail, aih)
    bul, buh, bil, bih = conv(edge, yul, yuh, yil, yih, zeros_f)
    gathered = _make_gather()(
        user, item_p, item_n, emb_user, emb_item,
        aul, auh, ail, aih, bul, buh, bil, bih, norm)
    s, t, a, c, lsm = _scores(gathered, W_user, b_user, W_item, b_item)
    out = _loss(s, t, a, c, lsm)
    return out[0, 0]


# speedup vs baseline: 10.3395x; 3.2264x over previous
import functools

import jax
import jax.numpy as jnp
from jax import lax
from jax.experimental import pallas as pl
from jax.experimental.pallas import tpu as pltpu
from jax.experimental.pallas import tpu_sc as plsc

NU = 50000
D = 64
HD = 32
E = 800000
B = 4096
NC = 2
NS = 16
CH = 128
NCHUNK = E // CH
QCONV = (NCHUNK + NS - 1) // NS
HCHUNK = NCHUNK // NC
QDEG = (HCHUNK + NS - 1) // NS
PDEG = 51200
DSLAB = PDEG // NS
NUP = 51200
RSLAB = NUP // NS
NFL = RSLAB // CH
NBUF = 6
RB2 = 2000
NB2 = NU // RB2
RB7 = 256
NB7 = B // RB7
ALPHA = 0.001
BETA = 0.001

_f32 = jnp.float32
_i32 = jnp.int32


def _sc_mesh():
    return plsc.VectorSubcoreMesh(
        core_axis_name="c", subcore_axis_name="s", num_cores=NC, num_subcores=NS
    )


def _sc_params():
    return pltpu.CompilerParams(use_tc_tiling_on_sc=False)


def _deg_body(edge, zeros_d, ones_c, out, acc, zbuf, obuf, idx, fbuf):
    c = lax.axis_index("c")
    s = lax.axis_index("s")
    pltpu.sync_copy(zeros_d, zbuf)
    pltpu.sync_copy(ones_c, obuf)
    pltpu.sync_copy(zbuf, acc.at[pl.ds(s * DSLAB, DSLAB)])
    plsc.subcore_barrier()
    start = s * QDEG
    n = jnp.clip(HCHUNK - start, 0, QDEG)

    def body(k, carry):
        e0 = (c * HCHUNK + start + k) * CH
        pltpu.sync_copy(edge.at[1, pl.ds(e0, CH)], idx)
        pltpu.sync_copy(obuf, acc.at[idx], add=True)
        return carry

    lax.fori_loop(0, n, body, 0)
    plsc.subcore_barrier()
    pltpu.sync_copy(acc.at[pl.ds(s * DSLAB, DSLAB)], fbuf)
    pltpu.sync_copy(fbuf, out.at[c, pl.ds(s * DSLAB, DSLAB)])


def _make_deg():
    return pl.kernel(
        _deg_body,
        compiler_params=_sc_params(),
        out_type=[jax.ShapeDtypeStruct((NC, PDEG), _f32)],
        mesh=_sc_mesh(),
        scratch_types=[
            pltpu.VMEM_SHARED((PDEG,), _f32),
            pltpu.VMEM((DSLAB,), _f32),
            pltpu.VMEM((CH,), _f32),
            pltpu.VMEM((CH,), _i32),
            pltpu.VMEM((DSLAB,), _f32),
        ],
    )


def _prep_body(deg_ref, eu_ref, ei_ref,
               norm_ref, xul_ref, xuh_ref, xil_ref, xih_ref):
    d = deg_ref[:, 0] + deg_ref[:, 1]
    nrm = lax.rsqrt(jnp.maximum(d, 1.0))
    norm_ref[...] = nrm[:, None]
    xu = eu_ref[...] * nrm[:, None]
    xul_ref[...] = xu[:, :HD]
    xuh_ref[...] = xu[:, HD:]
    xi = ei_ref[...] * nrm[:, None]
    xil_ref[...] = xi[:, :HD]
    xih_ref[...] = xi[:, HD:]


def _prep(deg_t, emb_user, emb_item):
    return pl.pallas_call(
        _prep_body,
        grid=(NB2,),
        in_specs=[
            pl.BlockSpec((RB2, NC), lambda k: (k, 0)),
            pl.BlockSpec((RB2, D), lambda k: (k, 0)),
            pl.BlockSpec((RB2, D), lambda k: (k, 0)),
        ],
        out_specs=[
            pl.BlockSpec((RB2, 1), lambda k: (k, 0)),
            pl.BlockSpec((RB2, HD), lambda k: (k, 0)),
            pl.BlockSpec((RB2, HD), lambda k: (k, 0)),
            pl.BlockSpec((RB2, HD), lambda k: (k, 0)),
            pl.BlockSpec((RB2, HD), lambda k: (k, 0)),
        ],
        out_shape=[
            jax.ShapeDtypeStruct((NUP, 1), _f32),
            jax.ShapeDtypeStruct((NUP, HD), _f32),
            jax.ShapeDtypeStruct((NUP, HD), _f32),
            jax.ShapeDtypeStruct((NUP, HD), _f32),
            jax.ShapeDtypeStruct((NUP, HD), _f32),
        ],
    )(deg_t, emb_user, emb_item)


def _conv_body(edge, a_lo, a_hi, b_lo, b_hi, zeros_f,
               oal, oah, obl, obh,
               acc, ibuf, rbuf, si, sg, sf):
    c = lax.axis_index("c")
    s = lax.axis_index("s")
    start = s * QCONV
    n = jnp.clip(NCHUNK - start, 0, QCONV)

    def idx_src(k):
        e0 = (start + k) * CH
        return edge.at[:, pl.ds(e0, CH)]

    def run_pass(in_lo, in_hi, out_lo, out_hi):
        pltpu.sync_copy(zeros_f, rbuf.at[0])
        for j in range(NFL):
            pltpu.sync_copy(rbuf.at[0], acc.at[pl.ds(s * RSLAB + j * CH, CH), :])
        plsc.subcore_barrier()

        def issue_idx(k, q):
            @pl.when(k < n)
            def _():
                pltpu.async_copy(idx_src(k), ibuf.at[q], si.at[q])

        def issue_gather(k, q, b):
            @pl.when(k < n)
            def _():
                pltpu.make_async_copy(idx_src(k), ibuf.at[q], si.at[q]).wait()

                @pl.when(c == 0)
                def _():
                    pltpu.async_copy(in_lo.at[ibuf.at[q, 0]], rbuf.at[b], sg.at[b])

                @pl.when(c == 1)
                def _():
                    pltpu.async_copy(in_hi.at[ibuf.at[q, 0]], rbuf.at[b], sg.at[b])

        def scatter_chunk(k, q, b):
            @pl.when(k < n)
            def _():
                @pl.when(c == 0)
                def _():
                    pltpu.make_async_copy(
                        in_lo.at[ibuf.at[q, 0]], rbuf.at[b], sg.at[b]).wait()

                @pl.when(c == 1)
                def _():
                    pltpu.make_async_copy(
                        in_hi.at[ibuf.at[q, 0]], rbuf.at[b], sg.at[b]).wait()

                pltpu.sync_copy(rbuf.at[b], acc.at[ibuf.at[q, 1]], add=True)

        for b in range(NBUF):
            issue_idx(jnp.int32(b), b)
        for b in range(NBUF):
            issue_idx(jnp.int32(NBUF + b), NBUF + b)
        for b in range(NBUF):
            issue_gather(jnp.int32(b), b, b)

        ng = (QCONV + NBUF - 1) // NBUF
        ng2 = (ng + 1) // 2

        def group2(g2, carry):
            g0 = g2 * 2
            for half in range(2):
                g = g0 + half
                for b in range(NBUF):
                    k = g * NBUF + b
                    q = half * NBUF + b
                    qn = (1 - half) * NBUF + b
                    scatter_chunk(k, q, b)
                    issue_idx(k + 2 * NBUF, q)
                    issue_gather(k + NBUF, qn, b)
            return carry

        lax.fori_loop(0, ng2, group2, 0)
        plsc.subcore_barrier()

        def out_dst(j, b):
            r0 = s * RSLAB + j * CH

            @pl.when(c == 0)
            def _():
                pltpu.async_copy(rbuf.at[b], out_lo.at[pl.ds(r0, CH), :], sf.at[b])

            @pl.when(c == 1)
            def _():
                pltpu.async_copy(rbuf.at[b], out_hi.at[pl.ds(r0, CH), :], sf.at[b])

        def out_wait(j, b):
            r0 = s * RSLAB + j * CH

            @pl.when(c == 0)
            def _():
                pltpu.make_async_copy(
                    rbuf.at[b], out_lo.at[pl.ds(r0, CH), :], sf.at[b]).wait()

            @pl.when(c == 1)
            def _():
                pltpu.make_async_copy(
                    rbuf.at[b], out_hi.at[pl.ds(r0, CH), :], sf.at[b]).wait()

        for j in range(NFL):
            b = j % NBUF
            if j >= NBUF:
                out_wait(j - NBUF, b)
            pltpu.sync_copy(acc.at[pl.ds(s * RSLAB + j * CH, CH), :], rbuf.at[b])
            out_dst(j, b)
        for j in range(max(0, NFL - NBUF), NFL):
            out_wait(j, j % NBUF)

    run_pass(a_lo, a_hi, oal, oah)
    run_pass(b_lo, b_hi, obl, obh)


def _make_conv():
    return pl.kernel(
        _conv_body,
        compiler_params=_sc_params(),
        out_type=[jax.ShapeDtypeStruct((NUP, HD), _f32)] * 4,
        mesh=_sc_mesh(),
        scratch_types=[
            pltpu.VMEM_SHARED((NUP, HD), _f32),
            pltpu.VMEM((2 * NBUF, 2, CH), _i32),
            pltpu.VMEM((NBUF, CH, HD), _f32),
            pltpu.SemaphoreType.DMA((2 * NBUF,)),
            pltpu.SemaphoreType.DMA((NBUF,)),
            pltpu.SemaphoreType.DMA((NBUF,)),
        ],
    )


def _scale_body(norm_ref, aul_ref, auh_ref, ail_ref, aih_ref,
                yul_ref, yuh_ref, yil_ref, yih_ref):
    n2 = norm_ref[...] * norm_ref[...]
    yul_ref[...] = aul_ref[...] * n2
    yuh_ref[...] = auh_ref[...] * n2
    yil_ref[...] = ail_ref[...] * n2
    yih_ref[...] = aih_ref[...] * n2


def _scale(norm, aul, auh, ail, aih):
    bs = pl.BlockSpec((RB2, HD), lambda k: (k, 0))
    return pl.pallas_call(
        _scale_body,
        grid=(NB2,),
        in_specs=[pl.BlockSpec((RB2, 1), lambda k: (k, 0)), bs, bs, bs, bs],
        out_specs=[bs, bs, bs, bs],
        out_shape=[jax.ShapeDtypeStruct((NUP, HD), _f32)] * 4,
    )(norm, aul, auh, ail, aih)


def _gather_body(user, item_p, item_n, emb_user, emb_item,
                 aul, auh, ail, aih, bul, buh, bil, bih, norm,
                 ue, ual, uah, ubl, ubh, un,
                 pe, pal, pah, pbl, pbh, pn,
                 ne, nal, nah, nbl, nbh, nn,
                 idx, rbuf, hbuf, nbuf):
    c = lax.axis_index("c")
    s = lax.axis_index("s")
    w = s * NC + c
    base = w * CH

    def do_set(iv, embt, tal, tah, tbl, tbh,
               oe, oal_, oah_, obl_, obh_, on_):
        pltpu.sync_copy(iv.at[pl.ds(base, CH)], idx)
        pltpu.sync_copy(embt.at[idx], rbuf)
        pltpu.sync_copy(rbuf, oe.at[pl.ds(base, CH), :])
        pltpu.sync_copy(tal.at[idx], hbuf)
        pltpu.sync_copy(hbuf, oal_.at[pl.ds(base, CH), :])
        pltpu.sync_copy(tah.at[idx], hbuf)
        pltpu.sync_copy(hbuf, oah_.at[pl.ds(base, CH), :])
        pltpu.sync_copy(tbl.at[idx], hbuf)
        pltpu.sync_copy(hbuf, obl_.at[pl.ds(base, CH), :])
        pltpu.sync_copy(tbh.at[idx], hbuf)
        pltpu.sync_copy(hbuf, obh_.at[pl.ds(base, CH), :])
        pltpu.sync_copy(norm.at[idx], nbuf)
        pltpu.sync_copy(nbuf, on_.at[pl.ds(base, CH), :])

    do_set(user, emb_user, ail, aih, bul, buh, ue, ual, uah, ubl, ubh, un)
    do_set(item_p, emb_item, aul, auh, bil, bih, pe, pal, pah, pbl, pbh, pn)
    do_set(item_n, emb_item, aul, auh, bil, bih, ne, nal, nah, nbl, nbh, nn)


def _make_gather():
    row = jax.ShapeDtypeStruct((B, D), _f32)
    half = jax.ShapeDtypeStruct((B, HD), _f32)
    vec = jax.ShapeDtypeStruct((B, 1), _f32)
    return pl.kernel(
        _gather_body,
        compiler_params=_sc_params(),
        out_type=[row, half, half, half, half, vec] * 3,
        mesh=_sc_mesh(),
        scratch_types=[
            pltpu.VMEM((CH,), _i32),
            pltpu.VMEM((CH, D), _f32),
            pltpu.VMEM((CH, HD), _f32),
            pltpu.VMEM((CH, 1), _f32),
        ],
    )


def _sig(x):
    return 1.0 / (1.0 + jnp.exp(-x))


def _combine(e_ref, al_ref, ah_ref, bl_ref, bh_ref, n_ref):
    nrm = n_ref[...]
    lo = (e_ref[...][:, :HD] + nrm * (al_ref[...] + bl_ref[...])) / 3.0
    hi = (e_ref[...][:, HD:] + nrm * (ah_ref[...] + bh_ref[...])) / 3.0
    return lo, hi


def _scores_body(ue, ual, uah, ubl, ubh, un,
                 pe, pal, pah, pbl, pbh, pn,
                 ne, nal, nah, nbl, nbh, nn,
                 wu_ref, bu_ref, wi_ref, bi_ref,
                 s_ref, t_ref, a_ref, c_ref, lsm_ref):
    uf_lo, uf_hi = _combine(ue, ual, uah, ubl, ubh, un)
    ip_lo, ip_hi = _combine(pe, pal, pah, pbl, pbh, pn)
    in_lo, in_hi = _combine(ne, nal, nah, nbl, nbh, nn)
    s_ref[...] = (jnp.sum(uf_lo * ip_lo, axis=1)
                  + jnp.sum(uf_hi * ip_hi, axis=1)) / float(D)
    t_ref[...] = (jnp.sum(uf_lo * in_lo, axis=1)
                  + jnp.sum(uf_hi * in_hi, axis=1)) / float(D)
    wu = wu_ref[0, :]
    wi = wi_ref[0, :]
    us = (jnp.sum(uf_lo * wu[None, :HD], axis=1)
          + jnp.sum(uf_hi * wu[None, HD:], axis=1) + bu_ref[0])
    pi = (jnp.sum(ip_lo * wi[None, :HD], axis=1)
          + jnp.sum(ip_hi * wi[None, HD:], axis=1) + bi_ref[0])
    ni = (jnp.sum(in_lo * wi[None, :HD], axis=1)
          + jnp.sum(in_hi * wi[None, HD:], axis=1) + bi_ref[0])
    sig_u = _sig(us)
    a_ref[...] = _sig(pi) * sig_u
    c_ref[...] = _sig(ni) * sig_u
    l_item = -jnp.mean(jnp.log(_sig(pi) + 1e-10) + jnp.log(1.0 - _sig(ni) + 1e-10))
    l_user = -jnp.mean(jnp.log(sig_u + 1e-10) + jnp.log(1.0 - sig_u + 1e-10))
    lsm_ref[...] = jnp.full((1, 1), ALPHA * l_item + BETA * l_user, _f32)


def _scores(gathered, W_user, b_user, W_item, b_item):
    full_row = pl.BlockSpec((B, D), lambda: (0, 0))
    full_half = pl.BlockSpec((B, HD), lambda: (0, 0))
    full_vec = pl.BlockSpec((B, 1), lambda: (0, 0))
    set_specs = [full_row, full_half, full_half, full_half, full_half, full_vec]
    return pl.pallas_call(
        _scores_body,
        in_specs=set_specs * 3 + [
            pl.BlockSpec((1, D), lambda: (0, 0)),
            pl.BlockSpec(memory_space=pltpu.SMEM),
            pl.BlockSpec((1, D), lambda: (0, 0)),
            pl.BlockSpec(memory_space=pltpu.SMEM),
        ],
        out_specs=[pl.BlockSpec((B,), lambda: (0,))] * 4
        + [pl.BlockSpec((1, 1), lambda: (0, 0))],
        out_shape=[jax.ShapeDtypeStruct((B,), _f32)] * 4
        + [jax.ShapeDtypeStruct((1, 1), _f32)],
    )(*gathered, W_user.reshape(1, D), b_user, W_item.reshape(1, D), b_item)


def _loss_body(s_ref, t_ref, a_ref, c_ref, lsm_ref, out_ref, acc_ref):
    i = pl.program_id(0)

    @pl.when(i == 0)
    def _():
        acc_ref[0] = 0.0

    x = s_ref[...][None, :] * a_ref[...][:, None]
    y = t_ref[...][None, :] * c_ref[...][:, None]
    part = jnp.sum(jnp.log(_sig(x) + 1e-10) + jnp.log(1.0 - _sig(y) + 1e-10))
    acc_ref[0] = acc_ref[0] + part

    @pl.when(i == NB7 - 1)
    def _():
        out_ref[...] = jnp.full(
            (1, 1), -(acc_ref[0] / float(B * B)) + lsm_ref[0, 0], _f32)


def _loss(s, t, a, c, lsm):
    return pl.pallas_call(
        _loss_body,
        grid=(NB7,),
        in_specs=[
            pl.BlockSpec((B,), lambda i: (0,)),
            pl.BlockSpec((B,), lambda i: (0,)),
            pl.BlockSpec((RB7,), lambda i: (i,)),
            pl.BlockSpec((RB7,), lambda i: (i,)),
            pl.BlockSpec((1, 1), lambda i: (0, 0)),
        ],
        out_specs=pl.BlockSpec((1, 1), lambda i: (0, 0)),
        out_shape=jax.ShapeDtypeStruct((1, 1), _f32),
        scratch_shapes=[pltpu.SMEM((1,), _f32)],
    )(s, t, a, c, lsm)


def kernel(emb_user, emb_item, W_user, b_user, W_item, b_item,
           user, item_p, item_n, edge_index):
    edge = edge_index.astype(_i32)
    user = user.astype(_i32)
    item_p = item_p.astype(_i32)
    item_n = item_n.astype(_i32)

    zeros_d = jnp.zeros((DSLAB,), _f32)
    ones_c = jnp.ones((CH,), _f32)
    zeros_f = jnp.zeros((CH, HD), _f32)

    (deg2,) = _make_deg()(edge, zeros_d, ones_c)
    deg_t = jnp.swapaxes(deg2, 0, 1)
    norm, xul, xuh, xil, xih = _prep(deg_t, emb_user, emb_item)
    conv = _make_conv()
    aul, auh, ail, aih = conv(edge, xul, xuh, xil, xih, zeros_f)
    yul, yuh, yil, yih = _scale(norm, aul, auh, ail, aih)
    bul, buh, bil, bih = conv(edge, yul, yuh, yil, yih, zeros_f)
    gathered = _make_gather()(
        user, item_p, item_n, emb_user, emb_item,
        aul, auh, ail, aih, bul, buh, bil, bih, norm)
    s, t, a, c, lsm = _scores(gathered, W_user, b_user, W_item, b_item)
    out = _loss(s, t, a, c, lsm)
    return out[0, 0]


# speedup vs baseline: 11.5695x; 1.1190x over previous
import functools

import jax
import jax.numpy as jnp
from jax import lax
from jax.experimental import pallas as pl
from jax.experimental.pallas import tpu as pltpu
from jax.experimental.pallas import tpu_sc as plsc

NU = 50000
D = 64
HD = 32
E = 800000
B = 4096
NC = 2
NS = 16
CH = 128
NCHUNK = E // CH
QCONV = (NCHUNK + NS - 1) // NS
HCHUNK = NCHUNK // NC
QDEG = (HCHUNK + NS - 1) // NS
PDEG = 51200
DSLAB = PDEG // NS
NUP = 51200
RSLAB = NUP // NS
NFL = RSLAB // CH
NBUF = 6
RB2 = 2048
NB2 = NUP // RB2
RB7 = 256
NB7 = B // RB7
RB7A = 512
NB7A = B // RB7A
ALPHA = 0.001
BETA = 0.001

_f32 = jnp.float32
_i32 = jnp.int32


def _sc_mesh():
    return plsc.VectorSubcoreMesh(
        core_axis_name="c", subcore_axis_name="s", num_cores=NC, num_subcores=NS
    )


def _sc_params():
    return pltpu.CompilerParams(use_tc_tiling_on_sc=False)


NBD = 12


def _deg_body(edge, zeros_d, ones_c, out, acc, zbuf, obuf, ibuf, fbuf, si, ss):
    c = lax.axis_index("c")
    s = lax.axis_index("s")
    pltpu.sync_copy(zeros_d, zbuf)
    pltpu.sync_copy(ones_c, obuf)
    pltpu.sync_copy(zbuf, acc.at[pl.ds(s * DSLAB, DSLAB)])
    plsc.subcore_barrier()
    start = s * QDEG
    n = jnp.clip(HCHUNK - start, 0, QDEG)

    def idx_src(k):
        e0 = (c * HCHUNK + start + k) * CH
        return edge.at[1, pl.ds(e0, CH)]

    def issue_idx(k, q):
        pltpu.async_copy(idx_src(k), ibuf.at[q], si.at[q])

    for q in range(NBD // 2):
        @pl.when(q < n)
        def _():
            issue_idx(jnp.int32(q), q)

    ngd = (QDEG + NBD - 1) // NBD

    def group(g, carry):
        j0 = g * NBD
        for q in range(NBD):
            j = j0 + q

            @pl.when(j < n)
            def _():
                pltpu.make_async_copy(idx_src(j), ibuf.at[q], si.at[q]).wait()

                @pl.when(j < n - NBD)
                def _():
                    pltpu.async_copy(obuf, acc.at[ibuf.at[q]], ss.at[q], add=True)

                @pl.when(j >= n - NBD)
                def _():
                    pltpu.sync_copy(obuf, acc.at[ibuf.at[q]], add=True)

            kk = j + NBD // 2
            qq = (q + NBD // 2) % NBD

            @pl.when(kk < n)
            def _():
                @pl.when(kk >= NBD)
                def _():
                    pltpu.make_async_copy(
                        obuf, acc.at[ibuf.at[qq]], ss.at[qq]).wait()

                issue_idx(kk, qq)
        return carry

    lax.fori_loop(0, ngd, group, 0)
    plsc.subcore_barrier()
    pltpu.sync_copy(acc.at[pl.ds(s * DSLAB, DSLAB)], fbuf)
    pltpu.sync_copy(fbuf, out.at[c, pl.ds(s * DSLAB, DSLAB)])


def _make_deg():
    return pl.kernel(
        _deg_body,
        compiler_params=_sc_params(),
        out_type=[jax.ShapeDtypeStruct((NC, PDEG), _f32)],
        mesh=_sc_mesh(),
        scratch_types=[
            pltpu.VMEM_SHARED((PDEG,), _f32),
            pltpu.VMEM((DSLAB,), _f32),
            pltpu.VMEM((CH,), _f32),
            pltpu.VMEM((NBD, CH), _i32),
            pltpu.VMEM((DSLAB,), _f32),
            pltpu.SemaphoreType.DMA((NBD,)),
            pltpu.SemaphoreType.DMA((NBD,)),
        ],
    )


def _prep_body(d0_ref, d1_ref, eu_ref, ei_ref,
               norm_ref, xul_ref, xuh_ref, xil_ref, xih_ref):
    d = d0_ref[...] + d1_ref[...]
    nrm = lax.rsqrt(jnp.maximum(d, 1.0))
    norm_ref[...] = nrm[:, None]
    xu = eu_ref[...] * nrm[:, None]
    xul_ref[...] = xu[:, :HD]
    xuh_ref[...] = xu[:, HD:]
    xi = ei_ref[...] * nrm[:, None]
    xil_ref[...] = xi[:, :HD]
    xih_ref[...] = xi[:, HD:]


def _prep(deg_flat, emb_user, emb_item):
    return pl.pallas_call(
        _prep_body,
        grid=(NB2,),
        in_specs=[
            pl.BlockSpec((RB2,), lambda k: (k,)),
            pl.BlockSpec((RB2,), lambda k: (k + NB2,)),
            pl.BlockSpec((RB2, D), lambda k: (k, 0)),
            pl.BlockSpec((RB2, D), lambda k: (k, 0)),
        ],
        out_specs=[
            pl.BlockSpec((RB2, 1), lambda k: (k, 0)),
            pl.BlockSpec((RB2, HD), lambda k: (k, 0)),
            pl.BlockSpec((RB2, HD), lambda k: (k, 0)),
            pl.BlockSpec((RB2, HD), lambda k: (k, 0)),
            pl.BlockSpec((RB2, HD), lambda k: (k, 0)),
        ],
        out_shape=[
            jax.ShapeDtypeStruct((NUP, 1), _f32),
            jax.ShapeDtypeStruct((NUP, HD), _f32),
            jax.ShapeDtypeStruct((NUP, HD), _f32),
            jax.ShapeDtypeStruct((NUP, HD), _f32),
            jax.ShapeDtypeStruct((NUP, HD), _f32),
        ],
    )(deg_flat, deg_flat, emb_user, emb_item)


def _conv_body(edge, a_lo, a_hi, b_lo, b_hi, zeros_f,
               oal, oah, obl, obh,
               acc, ibuf, rbuf, si, sg, sf):
    c = lax.axis_index("c")
    s = lax.axis_index("s")
    start = s * QCONV
    n = jnp.clip(NCHUNK - start, 0, QCONV)

    def idx_src(k):
        e0 = (start + k) * CH
        return edge.at[:, pl.ds(e0, CH)]

    def run_pass(in_lo, in_hi, out_lo, out_hi):
        pltpu.sync_copy(zeros_f, rbuf.at[0])
        for j in range(NFL):
            pltpu.sync_copy(rbuf.at[0], acc.at[pl.ds(s * RSLAB + j * CH, CH), :])
        plsc.subcore_barrier()

        def issue_idx(k, q):
            @pl.when(k < n)
            def _():
                pltpu.async_copy(idx_src(k), ibuf.at[q], si.at[q])

        def issue_gather(k, q, b):
            @pl.when(k < n)
            def _():
                pltpu.make_async_copy(idx_src(k), ibuf.at[q], si.at[q]).wait()

                @pl.when(c == 0)
                def _():
                    pltpu.async_copy(in_lo.at[ibuf.at[q, 0]], rbuf.at[b], sg.at[b])

                @pl.when(c == 1)
                def _():
                    pltpu.async_copy(in_hi.at[ibuf.at[q, 0]], rbuf.at[b], sg.at[b])

        def scatter_chunk(k, q, b):
            @pl.when(k < n)
            def _():
                @pl.when(c == 0)
                def _():
                    pltpu.make_async_copy(
                        in_lo.at[ibuf.at[q, 0]], rbuf.at[b], sg.at[b]).wait()

                @pl.when(c == 1)
                def _():
                    pltpu.make_async_copy(
                        in_hi.at[ibuf.at[q, 0]], rbuf.at[b], sg.at[b]).wait()

                pltpu.sync_copy(rbuf.at[b], acc.at[ibuf.at[q, 1]], add=True)

        for b in range(NBUF):
            issue_idx(jnp.int32(b), b)
        for b in range(NBUF):
            issue_idx(jnp.int32(NBUF + b), NBUF + b)
        for b in range(NBUF):
            issue_gather(jnp.int32(b), b, b)

        ng = (QCONV + NBUF - 1) // NBUF
        ng2 = (ng + 1) // 2

        def group2(g2, carry):
            g0 = g2 * 2
            for half in range(2):
                g = g0 + half
                for b in range(NBUF):
                    k = g * NBUF + b
                    q = half * NBUF + b
                    qn = (1 - half) * NBUF + b
                    scatter_chunk(k, q, b)
                    issue_idx(k + 2 * NBUF, q)
                    issue_gather(k + NBUF, qn, b)
            return carry

        lax.fori_loop(0, ng2, group2, 0)
        plsc.subcore_barrier()

        def out_dst(j, b):
            r0 = s * RSLAB + j * CH

            @pl.when(c == 0)
            def _():
                pltpu.async_copy(rbuf.at[b], out_lo.at[pl.ds(r0, CH), :], sf.at[b])

            @pl.when(c == 1)
            def _():
                pltpu.async_copy(rbuf.at[b], out_hi.at[pl.ds(r0, CH), :], sf.at[b])

        def out_wait(j, b):
            r0 = s * RSLAB + j * CH

            @pl.when(c == 0)
            def _():
                pltpu.make_async_copy(
                    rbuf.at[b], out_lo.at[pl.ds(r0, CH), :], sf.at[b]).wait()

            @pl.when(c == 1)
            def _():
                pltpu.make_async_copy(
                    rbuf.at[b], out_hi.at[pl.ds(r0, CH), :], sf.at[b]).wait()

        for j in range(NFL):
            b = j % NBUF
            if j >= NBUF:
                out_wait(j - NBUF, b)
            pltpu.sync_copy(acc.at[pl.ds(s * RSLAB + j * CH, CH), :], rbuf.at[b])
            out_dst(j, b)
        for j in range(max(0, NFL - NBUF), NFL):
            out_wait(j, j % NBUF)

    run_pass(a_lo, a_hi, oal, oah)
    run_pass(b_lo, b_hi, obl, obh)


def _make_conv():
    return pl.kernel(
        _conv_body,
        compiler_params=_sc_params(),
        out_type=[jax.ShapeDtypeStruct((NUP, HD), _f32)] * 4,
        mesh=_sc_mesh(),
        scratch_types=[
            pltpu.VMEM_SHARED((NUP, HD), _f32),
            pltpu.VMEM((2 * NBUF, 2, CH), _i32),
            pltpu.VMEM((NBUF, CH, HD), _f32),
            pltpu.SemaphoreType.DMA((2 * NBUF,)),
            pltpu.SemaphoreType.DMA((NBUF,)),
            pltpu.SemaphoreType.DMA((NBUF,)),
        ],
    )


def _scale_body(norm_ref, aul_ref, auh_ref, ail_ref, aih_ref,
                yul_ref, yuh_ref, yil_ref, yih_ref):
    n2 = norm_ref[...] * norm_ref[...]
    yul_ref[...] = aul_ref[...] * n2
    yuh_ref[...] = auh_ref[...] * n2
    yil_ref[...] = ail_ref[...] * n2
    yih_ref[...] = aih_ref[...] * n2


def _scale(norm, aul, auh, ail, aih):
    bs = pl.BlockSpec((RB2, HD), lambda k: (k, 0))
    return pl.pallas_call(
        _scale_body,
        grid=(NB2,),
        in_specs=[pl.BlockSpec((RB2, 1), lambda k: (k, 0)), bs, bs, bs, bs],
        out_specs=[bs, bs, bs, bs],
        out_shape=[jax.ShapeDtypeStruct((NUP, HD), _f32)] * 4,
    )(norm, aul, auh, ail, aih)


def _gather_body(user, item_p, item_n,
                 xul, xuh, xil, xih,
                 aul, auh, ail, aih, bul, buh, bil, bih, norm,
                 uxl, uxh, ual, uah, ubl, ubh, un,
                 pxl, pxh, pal, pah, pbl, pbh, pn,
                 nxl, nxh, nal, nah, nbl, nbh, nn,
                 idx, hbuf, nbuf):
    c = lax.axis_index("c")
    s = lax.axis_index("s")
    w = s * NC + c
    base = w * CH

    def one(tab, out):
        pltpu.sync_copy(tab.at[idx], hbuf)
        pltpu.sync_copy(hbuf, out.at[pl.ds(base, CH), :])

    def do_set(iv, txl, txh, tal, tah, tbl, tbh,
               oxl, oxh, oal_, oah_, obl_, obh_, on_):
        pltpu.sync_copy(iv.at[pl.ds(base, CH)], idx)
        one(txl, oxl)
        one(txh, oxh)
        one(tal, oal_)
        one(tah, oah_)
        one(tbl, obl_)
        one(tbh, obh_)
        pltpu.sync_copy(norm.at[idx], nbuf)
        pltpu.sync_copy(nbuf, on_.at[pl.ds(base, CH), :])

    do_set(user, xul, xuh, ail, aih, bul, buh,
           uxl, uxh, ual, uah, ubl, ubh, un)
    do_set(item_p, xil, xih, aul, auh, bil, bih,
           pxl, pxh, pal, pah, pbl, pbh, pn)
    do_set(item_n, xil, xih, aul, auh, bil, bih,
           nxl, nxh, nal, nah, nbl, nbh, nn)


def _make_gather():
    half = jax.ShapeDtypeStruct((B, HD), _f32)
    vec = jax.ShapeDtypeStruct((B, 1), _f32)
    return pl.kernel(
        _gather_body,
        compiler_params=_sc_params(),
        out_type=[half, half, half, half, half, half, vec] * 3,
        mesh=_sc_mesh(),
        scratch_types=[
            pltpu.VMEM((CH,), _i32),
            pltpu.VMEM((CH, HD), _f32),
            pltpu.VMEM((CH, 1), _f32),
        ],
    )


def _sig(x):
    return 1.0 / (1.0 + jnp.exp(-x))


def _combine(xl_ref, xh_ref, al_ref, ah_ref, bl_ref, bh_ref, n_ref):
    nrm = n_ref[...]
    inv = 1.0 / nrm
    lo = (xl_ref[...] * inv + nrm * (al_ref[...] + bl_ref[...])) / 3.0
    hi = (xh_ref[...] * inv + nrm * (ah_ref[...] + bh_ref[...])) / 3.0
    return lo, hi


def _scores_body(uxl, uxh, ual, uah, ubl, ubh, un,
                 pxl, pxh, pal, pah, pbl, pbh, pn,
                 nxl, nxh, nal, nah, nbl, nbh, nn,
                 wu_ref, bu_ref, wi_ref, bi_ref,
                 s_ref, t_ref, a_ref, c_ref, lsm_ref, acc_ref):
    i = pl.program_id(0)
    uf_lo, uf_hi = _combine(uxl, uxh, ual, uah, ubl, ubh, un)
    ip_lo, ip_hi = _combine(pxl, pxh, pal, pah, pbl, pbh, pn)
    in_lo, in_hi = _combine(nxl, nxh, nal, nah, nbl, nbh, nn)
    s_ref[...] = (jnp.sum(uf_lo * ip_lo, axis=1)
                  + jnp.sum(uf_hi * ip_hi, axis=1)) / float(D)
    t_ref[...] = (jnp.sum(uf_lo * in_lo, axis=1)
                  + jnp.sum(uf_hi * in_hi, axis=1)) / float(D)
    wu = wu_ref[0, :]
    wi = wi_ref[0, :]
    us = (jnp.sum(uf_lo * wu[None, :HD], axis=1)
          + jnp.sum(uf_hi * wu[None, HD:], axis=1) + bu_ref[0])
    pi = (jnp.sum(ip_lo * wi[None, :HD], axis=1)
          + jnp.sum(ip_hi * wi[None, HD:], axis=1) + bi_ref[0])
    ni = (jnp.sum(in_lo * wi[None, :HD], axis=1)
          + jnp.sum(in_hi * wi[None, HD:], axis=1) + bi_ref[0])
    sig_u = _sig(us)
    a_ref[...] = _sig(pi) * sig_u
    c_ref[...] = _sig(ni) * sig_u

    @pl.when(i == 0)
    def _():
        acc_ref[0] = 0.0
        acc_ref[1] = 0.0

    acc_ref[0] = acc_ref[0] + jnp.sum(
        jnp.log(_sig(pi) + 1e-10) + jnp.log(1.0 - _sig(ni) + 1e-10))
    acc_ref[1] = acc_ref[1] + jnp.sum(
        jnp.log(sig_u + 1e-10) + jnp.log(1.0 - sig_u + 1e-10))

    @pl.when(i == NB7A - 1)
    def _():
        lsm_ref[...] = jnp.full(
            (1, 1),
            ALPHA * (-acc_ref[0] / float(B)) + BETA * (-acc_ref[1] / float(B)),
            _f32)


def _scores(gathered, W_user, b_user, W_item, b_item):
    blk_half = pl.BlockSpec((RB7A, HD), lambda i: (i, 0))
    blk_vec = pl.BlockSpec((RB7A, 1), lambda i: (i, 0))
    set_specs = [blk_half] * 6 + [blk_vec]
    return pl.pallas_call(
        _scores_body,
        grid=(NB7A,),
        in_specs=set_specs * 3 + [
            pl.BlockSpec((1, D), lambda i: (0, 0)),
            pl.BlockSpec(memory_space=pltpu.SMEM),
            pl.BlockSpec((1, D), lambda i: (0, 0)),
            pl.BlockSpec(memory_space=pltpu.SMEM),
        ],
        out_specs=[pl.BlockSpec((RB7A,), lambda i: (i,))] * 4
        + [pl.BlockSpec((1, 1), lambda i: (0, 0))],
        out_shape=[jax.ShapeDtypeStruct((B,), _f32)] * 4
        + [jax.ShapeDtypeStruct((1, 1), _f32)],
        scratch_shapes=[pltpu.SMEM((2,), _f32)],
    )(*gathered, W_user.reshape(1, D), b_user, W_item.reshape(1, D), b_item)


def _loss_body(s_ref, t_ref, a_ref, c_ref, lsm_ref, out_ref, acc_ref):
    i = pl.program_id(0)

    @pl.when(i == 0)
    def _():
        acc_ref[0] = 0.0

    x = s_ref[...][None, :] * a_ref[...][:, None]
    y = t_ref[...][None, :] * c_ref[...][:, None]
    part = jnp.sum(jnp.log(_sig(x) + 1e-10) + jnp.log(1.0 - _sig(y) + 1e-10))
    acc_ref[0] = acc_ref[0] + part

    @pl.when(i == NB7 - 1)
    def _():
        out_ref[...] = jnp.full(
            (1, 1), -(acc_ref[0] / float(B * B)) + lsm_ref[0, 0], _f32)


def _loss(s, t, a, c, lsm):
    return pl.pallas_call(
        _loss_body,
        grid=(NB7,),
        in_specs=[
            pl.BlockSpec((B,), lambda i: (0,)),
            pl.BlockSpec((B,), lambda i: (0,)),
            pl.BlockSpec((RB7,), lambda i: (i,)),
            pl.BlockSpec((RB7,), lambda i: (i,)),
            pl.BlockSpec((1, 1), lambda i: (0, 0)),
        ],
        out_specs=pl.BlockSpec((1, 1), lambda i: (0, 0)),
        out_shape=jax.ShapeDtypeStruct((1, 1), _f32),
        scratch_shapes=[pltpu.SMEM((1,), _f32)],
    )(s, t, a, c, lsm)


def kernel(emb_user, emb_item, W_user, b_user, W_item, b_item,
           user, item_p, item_n, edge_index):
    edge = edge_index.astype(_i32)
    user = user.astype(_i32)
    item_p = item_p.astype(_i32)
    item_n = item_n.astype(_i32)

    zeros_d = jnp.zeros((DSLAB,), _f32)
    ones_c = jnp.ones((CH,), _f32)
    zeros_f = jnp.zeros((CH, HD), _f32)

    (deg2,) = _make_deg()(edge, zeros_d, ones_c)
    norm, xul, xuh, xil, xih = _prep(deg2.reshape(-1), emb_user, emb_item)
    conv = _make_conv()
    aul, auh, ail, aih = conv(edge, xul, xuh, xil, xih, zeros_f)
    yul, yuh, yil, yih = _scale(norm, aul, auh, ail, aih)
    bul, buh, bil, bih = conv(edge, yul, yuh, yil, yih, zeros_f)
    gathered = _make_gather()(
        user, item_p, item_n, xul, xuh, xil, xih,
        aul, auh, ail, aih, bul, buh, bil, bih, norm)
    s, t, a, c, lsm = _scores(gathered, W_user, b_user, W_item, b_item)
    out = _loss(s, t, a, c, lsm)
    return out[0, 0]


# speedup vs baseline: 13.1749x; 1.1388x over previous
import functools

import jax
import jax.numpy as jnp
from jax import lax
from jax.experimental import pallas as pl
from jax.experimental.pallas import tpu as pltpu
from jax.experimental.pallas import tpu_sc as plsc

NU = 50000
D = 64
HD = 32
E = 800000
B = 4096
NC = 2
NS = 16
CH = 128
NCHUNK = E // CH
QCONV = (NCHUNK + NS - 1) // NS
HCHUNK = NCHUNK // NC
QDEG = (HCHUNK + NS - 1) // NS
PDEG = 51200
DSLAB = PDEG // NS
NUP = 51200
RSLAB = NUP // NS
NFL = RSLAB // CH
NBUF = 6
RB2 = 2048
NB2 = NUP // RB2
RB7 = 256
NB7 = B // RB7
RB7A = 512
NB7A = B // RB7A
ALPHA = 0.001
BETA = 0.001

_f32 = jnp.float32
_i32 = jnp.int32


def _sc_mesh():
    return plsc.VectorSubcoreMesh(
        core_axis_name="c", subcore_axis_name="s", num_cores=NC, num_subcores=NS
    )


def _sc_params():
    return pltpu.CompilerParams(use_tc_tiling_on_sc=False)


NBD = 12


def _deg_body(edge, zeros_d, ones_c, out, acc, zbuf, obuf, ibuf, fbuf, si, ss):
    c = lax.axis_index("c")
    s = lax.axis_index("s")
    pltpu.sync_copy(zeros_d, zbuf)
    pltpu.sync_copy(ones_c, obuf)
    pltpu.sync_copy(zbuf, acc.at[pl.ds(s * DSLAB, DSLAB)])
    plsc.subcore_barrier()
    start = s * QDEG
    n = jnp.clip(HCHUNK - start, 0, QDEG)

    def idx_src(k):
        e0 = (c * HCHUNK + start + k) * CH
        return edge.at[1, pl.ds(e0, CH)]

    def issue_idx(k, q):
        pltpu.async_copy(idx_src(k), ibuf.at[q], si.at[q])

    for q in range(NBD // 2):
        @pl.when(q < n)
        def _():
            issue_idx(jnp.int32(q), q)

    ngd = (QDEG + NBD - 1) // NBD

    def group(g, carry):
        j0 = g * NBD
        for q in range(NBD):
            j = j0 + q

            @pl.when(j < n)
            def _():
                pltpu.make_async_copy(idx_src(j), ibuf.at[q], si.at[q]).wait()

                @pl.when(j < n - NBD)
                def _():
                    pltpu.async_copy(obuf, acc.at[ibuf.at[q]], ss.at[q], add=True)

                @pl.when(j >= n - NBD)
                def _():
                    pltpu.sync_copy(obuf, acc.at[ibuf.at[q]], add=True)

            kk = j + NBD // 2
            qq = (q + NBD // 2) % NBD

            @pl.when(kk < n)
            def _():
                @pl.when(kk >= NBD)
                def _():
                    pltpu.make_async_copy(
                        obuf, acc.at[ibuf.at[qq]], ss.at[qq]).wait()

                issue_idx(kk, qq)
        return carry

    lax.fori_loop(0, ngd, group, 0)
    plsc.subcore_barrier()
    pltpu.sync_copy(acc.at[pl.ds(s * DSLAB, DSLAB)], fbuf)
    pltpu.sync_copy(fbuf, out.at[c, pl.ds(s * DSLAB, DSLAB)])


def _make_deg():
    return pl.kernel(
        _deg_body,
        compiler_params=_sc_params(),
        out_type=[jax.ShapeDtypeStruct((NC, PDEG), _f32)],
        mesh=_sc_mesh(),
        scratch_types=[
            pltpu.VMEM_SHARED((PDEG,), _f32),
            pltpu.VMEM((DSLAB,), _f32),
            pltpu.VMEM((CH,), _f32),
            pltpu.VMEM((NBD, CH), _i32),
            pltpu.VMEM((DSLAB,), _f32),
            pltpu.SemaphoreType.DMA((NBD,)),
            pltpu.SemaphoreType.DMA((NBD,)),
        ],
    )


def _prep_body(d0_ref, d1_ref, eu_ref, ei_ref,
               norm_ref, xul_ref, xuh_ref, xil_ref, xih_ref):
    d = d0_ref[...] + d1_ref[...]
    nrm = lax.rsqrt(jnp.maximum(d, 1.0))
    norm_ref[...] = nrm[:, None]
    xu = eu_ref[...] * nrm[:, None]
    xul_ref[...] = xu[:, :HD]
    xuh_ref[...] = xu[:, HD:]
    xi = ei_ref[...] * nrm[:, None]
    xil_ref[...] = xi[:, :HD]
    xih_ref[...] = xi[:, HD:]


def _prep(deg_flat, emb_user, emb_item):
    return pl.pallas_call(
        _prep_body,
        grid=(NB2,),
        in_specs=[
            pl.BlockSpec((RB2,), lambda k: (k,)),
            pl.BlockSpec((RB2,), lambda k: (k + NB2,)),
            pl.BlockSpec((RB2, D), lambda k: (k, 0)),
            pl.BlockSpec((RB2, D), lambda k: (k, 0)),
        ],
        out_specs=[
            pl.BlockSpec((RB2, 1), lambda k: (k, 0)),
            pl.BlockSpec((RB2, HD), lambda k: (k, 0)),
            pl.BlockSpec((RB2, HD), lambda k: (k, 0)),
            pl.BlockSpec((RB2, HD), lambda k: (k, 0)),
            pl.BlockSpec((RB2, HD), lambda k: (k, 0)),
        ],
        out_shape=[
            jax.ShapeDtypeStruct((NUP, 1), _f32),
            jax.ShapeDtypeStruct((NUP, HD), _f32),
            jax.ShapeDtypeStruct((NUP, HD), _f32),
            jax.ShapeDtypeStruct((NUP, HD), _f32),
            jax.ShapeDtypeStruct((NUP, HD), _f32),
        ],
    )(deg_flat, deg_flat, emb_user, emb_item)


def _conv_body(edge, a_lo, a_hi, zeros_f,
               oal, oah,
               acc, ibuf, rbuf, si, sg, sf):
    c = lax.axis_index("c")
    s = lax.axis_index("s")
    start = s * QCONV
    n = jnp.clip(NCHUNK - start, 0, QCONV)

    def idx_src(k):
        e0 = (start + k) * CH
        return edge.at[:, pl.ds(e0, CH)]

    def run_pass(in_lo, in_hi, out_lo, out_hi):
        pltpu.sync_copy(zeros_f, rbuf.at[0])
        for j in range(NFL):
            pltpu.sync_copy(rbuf.at[0], acc.at[pl.ds(s * RSLAB + j * CH, CH), :])
        plsc.subcore_barrier()

        def issue_idx(k, q):
            @pl.when(k < n)
            def _():
                pltpu.async_copy(idx_src(k), ibuf.at[q], si.at[q])

        def issue_gather(k, q, b):
            @pl.when(k < n)
            def _():
                pltpu.make_async_copy(idx_src(k), ibuf.at[q], si.at[q]).wait()

                @pl.when(c == 0)
                def _():
                    pltpu.async_copy(in_lo.at[ibuf.at[q, 0]], rbuf.at[b], sg.at[b])

                @pl.when(c == 1)
                def _():
                    pltpu.async_copy(in_hi.at[ibuf.at[q, 0]], rbuf.at[b], sg.at[b])

        def scatter_chunk(k, q, b):
            @pl.when(k < n)
            def _():
                @pl.when(c == 0)
                def _():
                    pltpu.make_async_copy(
                        in_lo.at[ibuf.at[q, 0]], rbuf.at[b], sg.at[b]).wait()

                @pl.when(c == 1)
                def _():
                    pltpu.make_async_copy(
                        in_hi.at[ibuf.at[q, 0]], rbuf.at[b], sg.at[b]).wait()

                pltpu.sync_copy(rbuf.at[b], acc.at[ibuf.at[q, 1]], add=True)

        for b in range(NBUF):
            issue_idx(jnp.int32(b), b)
        for b in range(NBUF):
            issue_idx(jnp.int32(NBUF + b), NBUF + b)
        for b in range(NBUF):
            issue_gather(jnp.int32(b), b, b)

        ng = (QCONV + NBUF - 1) // NBUF
        ng2 = (ng + 1) // 2

        def group2(g2, carry):
            g0 = g2 * 2
            for half in range(2):
                g = g0 + half
                for b in range(NBUF):
                    k = g * NBUF + b
                    q = half * NBUF + b
                    qn = (1 - half) * NBUF + b
                    scatter_chunk(k, q, b)
                    issue_idx(k + 2 * NBUF, q)
                    issue_gather(k + NBUF, qn, b)
            return carry

        lax.fori_loop(0, ng2, group2, 0)
        plsc.subcore_barrier()

        def out_dst(j, b):
            r0 = s * RSLAB + j * CH

            @pl.when(c == 0)
            def _():
                pltpu.async_copy(rbuf.at[b], out_lo.at[pl.ds(r0, CH), :], sf.at[b])

            @pl.when(c == 1)
            def _():
                pltpu.async_copy(rbuf.at[b], out_hi.at[pl.ds(r0, CH), :], sf.at[b])

        def out_wait(j, b):
            r0 = s * RSLAB + j * CH

            @pl.when(c == 0)
            def _():
                pltpu.make_async_copy(
                    rbuf.at[b], out_lo.at[pl.ds(r0, CH), :], sf.at[b]).wait()

            @pl.when(c == 1)
            def _():
                pltpu.make_async_copy(
                    rbuf.at[b], out_hi.at[pl.ds(r0, CH), :], sf.at[b]).wait()

        for j in range(NFL):
            b = j % NBUF
            if j >= NBUF:
                out_wait(j - NBUF, b)
            pltpu.sync_copy(acc.at[pl.ds(s * RSLAB + j * CH, CH), :], rbuf.at[b])
            out_dst(j, b)
        for j in range(max(0, NFL - NBUF), NFL):
            out_wait(j, j % NBUF)

    run_pass(a_lo, a_hi, oal, oah)


def _make_conv():
    return pl.kernel(
        _conv_body,
        compiler_params=_sc_params(),
        out_type=[jax.ShapeDtypeStruct((NUP, HD), _f32)] * 2,
        mesh=_sc_mesh(),
        scratch_types=[
            pltpu.VMEM_SHARED((NUP, HD), _f32),
            pltpu.VMEM((2 * NBUF, 2, CH), _i32),
            pltpu.VMEM((NBUF, CH, HD), _f32),
            pltpu.SemaphoreType.DMA((2 * NBUF,)),
            pltpu.SemaphoreType.DMA((NBUF,)),
            pltpu.SemaphoreType.DMA((NBUF,)),
        ],
    )


def _scale_body(norm_ref, al_ref, ah_ref, yl_ref, yh_ref):
    n2 = norm_ref[...] * norm_ref[...]
    yl_ref[...] = al_ref[...] * n2
    yh_ref[...] = ah_ref[...] * n2


def _scale(norm, al, ah):
    bs = pl.BlockSpec((RB2, HD), lambda k: (k, 0))
    return pl.pallas_call(
        _scale_body,
        grid=(NB2,),
        in_specs=[pl.BlockSpec((RB2, 1), lambda k: (k, 0)), bs, bs],
        out_specs=[bs, bs],
        out_shape=[jax.ShapeDtypeStruct((NUP, HD), _f32)] * 2,
    )(norm, al, ah)


def _gather_body(user, item_p, item_n,
                 xul, xuh, xil, xih,
                 aul, auh, ail, aih, bul, buh, bil, bih, norm,
                 uxl, uxh, ual, uah, ubl, ubh, un,
                 pxl, pxh, pal, pah, pbl, pbh, pn,
                 nxl, nxh, nal, nah, nbl, nbh, nn,
                 idx, hbuf, nbuf):
    c = lax.axis_index("c")
    s = lax.axis_index("s")
    w = s * NC + c
    base = w * CH

    def one(tab, out):
        pltpu.sync_copy(tab.at[idx], hbuf)
        pltpu.sync_copy(hbuf, out.at[pl.ds(base, CH), :])

    def do_set(iv, txl, txh, tal, tah, tbl, tbh,
               oxl, oxh, oal_, oah_, obl_, obh_, on_):
        pltpu.sync_copy(iv.at[pl.ds(base, CH)], idx)
        one(txl, oxl)
        one(txh, oxh)
        one(tal, oal_)
        one(tah, oah_)
        one(tbl, obl_)
        one(tbh, obh_)
        pltpu.sync_copy(norm.at[idx], nbuf)
        pltpu.sync_copy(nbuf, on_.at[pl.ds(base, CH), :])

    do_set(user, xul, xuh, ail, aih, bul, buh,
           uxl, uxh, ual, uah, ubl, ubh, un)
    do_set(item_p, xil, xih, aul, auh, bil, bih,
           pxl, pxh, pal, pah, pbl, pbh, pn)
    do_set(item_n, xil, xih, aul, auh, bil, bih,
           nxl, nxh, nal, nah, nbl, nbh, nn)


def _make_gather():
    half = jax.ShapeDtypeStruct((B, HD), _f32)
    vec = jax.ShapeDtypeStruct((B, 1), _f32)
    return pl.kernel(
        _gather_body,
        compiler_params=_sc_params(),
        out_type=[half, half, half, half, half, half, vec] * 3,
        mesh=_sc_mesh(),
        scratch_types=[
            pltpu.VMEM((CH,), _i32),
            pltpu.VMEM((CH, HD), _f32),
            pltpu.VMEM((CH, 1), _f32),
        ],
    )


def _sig(x):
    return 1.0 / (1.0 + jnp.exp(-x))


def _combine(xl_ref, xh_ref, al_ref, ah_ref, bl_ref, bh_ref, n_ref):
    nrm = n_ref[...]
    inv = 1.0 / nrm
    lo = (xl_ref[...] * inv + nrm * (al_ref[...] + bl_ref[...])) / 3.0
    hi = (xh_ref[...] * inv + nrm * (ah_ref[...] + bh_ref[...])) / 3.0
    return lo, hi


def _scores_body(uxl, uxh, ual, uah, ubl, ubh, un,
                 pxl, pxh, pal, pah, pbl, pbh, pn,
                 nxl, nxh, nal, nah, nbl, nbh, nn,
                 wu_ref, bu_ref, wi_ref, bi_ref,
                 s_ref, t_ref, a_ref, c_ref, lsm_ref, acc_ref):
    i = pl.program_id(0)
    uf_lo, uf_hi = _combine(uxl, uxh, ual, uah, ubl, ubh, un)
    ip_lo, ip_hi = _combine(pxl, pxh, pal, pah, pbl, pbh, pn)
    in_lo, in_hi = _combine(nxl, nxh, nal, nah, nbl, nbh, nn)
    s_ref[...] = (jnp.sum(uf_lo * ip_lo, axis=1)
                  + jnp.sum(uf_hi * ip_hi, axis=1)) / float(D)
    t_ref[...] = (jnp.sum(uf_lo * in_lo, axis=1)
                  + jnp.sum(uf_hi * in_hi, axis=1)) / float(D)
    wu = wu_ref[0, :]
    wi = wi_ref[0, :]
    us = (jnp.sum(uf_lo * wu[None, :HD], axis=1)
          + jnp.sum(uf_hi * wu[None, HD:], axis=1) + bu_ref[0])
    pi = (jnp.sum(ip_lo * wi[None, :HD], axis=1)
          + jnp.sum(ip_hi * wi[None, HD:], axis=1) + bi_ref[0])
    ni = (jnp.sum(in_lo * wi[None, :HD], axis=1)
          + jnp.sum(in_hi * wi[None, HD:], axis=1) + bi_ref[0])
    sig_u = _sig(us)
    a_ref[...] = _sig(pi) * sig_u
    c_ref[...] = _sig(ni) * sig_u

    @pl.when(i == 0)
    def _():
        acc_ref[0] = 0.0
        acc_ref[1] = 0.0

    acc_ref[0] = acc_ref[0] + jnp.sum(
        jnp.log(_sig(pi) + 1e-10) + jnp.log(1.0 - _sig(ni) + 1e-10))
    acc_ref[1] = acc_ref[1] + jnp.sum(
        jnp.log(sig_u + 1e-10) + jnp.log(1.0 - sig_u + 1e-10))

    @pl.when(i == NB7A - 1)
    def _():
        lsm_ref[...] = jnp.full(
            (1, 1),
            ALPHA * (-acc_ref[0] / float(B)) + BETA * (-acc_ref[1] / float(B)),
            _f32)


def _scores(gathered, W_user, b_user, W_item, b_item):
    blk_half = pl.BlockSpec((RB7A, HD), lambda i: (i, 0))
    blk_vec = pl.BlockSpec((RB7A, 1), lambda i: (i, 0))
    set_specs = [blk_half] * 6 + [blk_vec]
    return pl.pallas_call(
        _scores_body,
        grid=(NB7A,),
        in_specs=set_specs * 3 + [
            pl.BlockSpec((1, D), lambda i: (0, 0)),
            pl.BlockSpec(memory_space=pltpu.SMEM),
            pl.BlockSpec((1, D), lambda i: (0, 0)),
            pl.BlockSpec(memory_space=pltpu.SMEM),
        ],
        out_specs=[pl.BlockSpec((RB7A,), lambda i: (i,))] * 4
        + [pl.BlockSpec((1, 1), lambda i: (0, 0))],
        out_shape=[jax.ShapeDtypeStruct((B,), _f32)] * 4
        + [jax.ShapeDtypeStruct((1, 1), _f32)],
        scratch_shapes=[pltpu.SMEM((2,), _f32)],
    )(*gathered, W_user.reshape(1, D), b_user, W_item.reshape(1, D), b_item)


def _loss_body(s_ref, t_ref, a_ref, c_ref, lsm_ref, out_ref, acc_ref):
    i = pl.program_id(0)

    @pl.when(i == 0)
    def _():
        acc_ref[0] = 0.0

    x = s_ref[...][None, :] * a_ref[...][:, None]
    y = t_ref[...][None, :] * c_ref[...][:, None]
    part = jnp.sum(jnp.log(_sig(x) + 1e-10) + jnp.log(1.0 - _sig(y) + 1e-10))
    acc_ref[0] = acc_ref[0] + part

    @pl.when(i == NB7 - 1)
    def _():
        out_ref[...] = jnp.full(
            (1, 1), -(acc_ref[0] / float(B * B)) + lsm_ref[0, 0], _f32)


def _loss(s, t, a, c, lsm):
    return pl.pallas_call(
        _loss_body,
        grid=(NB7,),
        in_specs=[
            pl.BlockSpec((B,), lambda i: (0,)),
            pl.BlockSpec((B,), lambda i: (0,)),
            pl.BlockSpec((RB7,), lambda i: (i,)),
            pl.BlockSpec((RB7,), lambda i: (i,)),
            pl.BlockSpec((1, 1), lambda i: (0, 0)),
        ],
        out_specs=pl.BlockSpec((1, 1), lambda i: (0, 0)),
        out_shape=jax.ShapeDtypeStruct((1, 1), _f32),
        scratch_shapes=[pltpu.SMEM((1,), _f32)],
    )(s, t, a, c, lsm)


def kernel(emb_user, emb_item, W_user, b_user, W_item, b_item,
           user, item_p, item_n, edge_index):
    edge = edge_index.astype(_i32)
    user = user.astype(_i32)
    item_p = item_p.astype(_i32)
    item_n = item_n.astype(_i32)

    zeros_d = jnp.zeros((DSLAB,), _f32)
    ones_c = jnp.ones((CH,), _f32)
    zeros_f = jnp.zeros((CH, HD), _f32)

    (deg2,) = _make_deg()(edge, zeros_d, ones_c)
    norm, xul, xuh, xil, xih = _prep(deg2.reshape(-1), emb_user, emb_item)
    conv = _make_conv()
    aul, auh = conv(edge, xul, xuh, zeros_f)
    ail, aih = conv(edge, xil, xih, zeros_f)
    yul, yuh = _scale(norm, aul, auh)
    yil, yih = _scale(norm, ail, aih)
    bul, buh = conv(edge, yul, yuh, zeros_f)
    bil, bih = conv(edge, yil, yih, zeros_f)
    gathered = _make_gather()(
        user, item_p, item_n, xul, xuh, xil, xih,
        aul, auh, ail, aih, bul, buh, bil, bih, norm)
    s, t, a, c, lsm = _scores(gathered, W_user, b_user, W_item, b_item)
    out = _loss(s, t, a, c, lsm)
    return out[0, 0]
